# baseline (speedup 1.0000x reference)
import functools

import jax
import jax.numpy as jnp
from jax import lax
from jax.experimental import pallas as pl
from jax.experimental.pallas import tpu as pltpu
from jax.experimental.pallas import tpu_sc as plsc

N = 10000
E = 320000
D = 128
DE = 16
H = 8
DH = 16

NC = 2
NS = 16
NW = NC * NS
PER_TILE = E // NW
C = 40
NCHUNK = PER_TILE // C
W = 144
NPAD = 10240
ROWS_PER_TILE = NPAD // NS


def _qkv_body(x_ref, wq_ref, wk_ref, wv_ref, q_ref, kv_ref):
    xb = x_ref[...]
    q_ref[...] = jnp.dot(xb, wq_ref[...], preferred_element_type=jnp.float32)
    kv_ref[:, :D] = jnp.dot(xb, wk_ref[...], preferred_element_type=jnp.float32)
    kv_ref[:, D:] = jnp.dot(xb, wv_ref[...], preferred_element_type=jnp.float32)


def _qkv(x, WQ, WK, WV):
    blk = 1000
    grid = (N // blk,)
    spec_x = pl.BlockSpec((blk, D), lambda i: (i, 0))
    spec_w = pl.BlockSpec((D, D), lambda i: (0, 0))
    return pl.pallas_call(
        _qkv_body,
        grid=grid,
        in_specs=[spec_x, spec_w, spec_w, spec_w],
        out_specs=[pl.BlockSpec((blk, D), lambda i: (i, 0)),
                   pl.BlockSpec((blk, 2 * D), lambda i: (i, 0))],
        out_shape=[jax.ShapeDtypeStruct((N, D), jnp.float32),
                   jax.ShapeDtypeStruct((N, 2 * D), jnp.float32)],
    )(x, WQ, WK, WV)


def _eproj_body(ea_ref, we_ref, eh_ref):
    eh_ref[...] = jnp.dot(ea_ref[...], we_ref[...],
                          preferred_element_type=jnp.float32)


def _eproj(edge_attr, WE):
    blk = 4000
    grid = (E // blk,)
    return pl.pallas_call(
        _eproj_body,
        grid=grid,
        in_specs=[pl.BlockSpec((blk, DE), lambda i: (i, 0)),
                  pl.BlockSpec((DE, D), lambda i: (0, 0))],
        out_specs=pl.BlockSpec((blk, D), lambda i: (i, 0)),
        out_shape=jax.ShapeDtypeStruct((E, D), jnp.float32),
    )(edge_attr, WE)


_mesh = plsc.VectorSubcoreMesh(core_axis_name="c", subcore_axis_name="s")

_GDN = lax.GatherDimensionNumbers(
    offset_dims=(), collapsed_slice_dims=(0,), start_index_map=(0,))


def _shuf(v, perm):
    return lax.gather(v, perm[:, None], _GDN, (1,),
                      mode=lax.GatherScatterMode.PROMISE_IN_BOUNDS)


@functools.partial(
    pl.kernel,
    out_type=jax.ShapeDtypeStruct((NC * NPAD, W), jnp.float32),
    mesh=_mesh,
    scratch_types=[
        pltpu.VMEM((C,), jnp.int32),
        pltpu.VMEM((C,), jnp.int32),
        pltpu.VMEM((C, 2 * D), jnp.float32),
        pltpu.VMEM((C, D), jnp.float32),
        pltpu.VMEM((C, D), jnp.float32),
        pltpu.VMEM((C, W), jnp.float32),
        pltpu.VMEM_SHARED((NPAD, W), jnp.float32),
        pltpu.SemaphoreType.DMA,
    ],
    compiler_params=pltpu.CompilerParams(use_tc_tiling_on_sc=False),
)
def _sc_attn(kvh, qh, eh, src, dst, out,
             src_v, dst_v, kvbuf, qbuf, ebuf, msgbuf, acc, sem):
    c = lax.axis_index("c")
    s = lax.axis_index("s")
    lane = lax.iota(jnp.int32, 16)
    zero16 = jnp.zeros((16,), jnp.float32)

    def zrow(r, carry):
        for j in range(W // 16):
            msgbuf[r, pl.ds(j * 16, 16)] = zero16
        return carry
    lax.fori_loop(0, C, zrow, 0)
    row0 = s * ROWS_PER_TILE
    for i in range(ROWS_PER_TILE // C):
        pltpu.sync_copy(msgbuf, acc.at[pl.ds(row0 + i * C, C)])
    plsc.subcore_barrier()

    tile_base = (c * NS + s) * PER_TILE

    def chunk_body(i, carry):
        base = tile_base + i * C
        pltpu.sync_copy(src.at[pl.ds(base, C)], src_v)
        pltpu.sync_copy(dst.at[pl.ds(base, C)], dst_v)
        cp1 = pltpu.async_copy(kvh.at[src_v], kvbuf, sem)
        cp2 = pltpu.async_copy(qh.at[dst_v], qbuf, sem)
        cp3 = pltpu.async_copy(eh.at[pl.ds(base, C)], ebuf, sem)
        cp1.wait(); cp2.wait(); cp3.wait()

        def merge(a, b, k):
            m = (lane & k) == 0
            pk = lane ^ k
            return (jnp.where(m, a, b) + jnp.where(m, _shuf(a, pk),
                                                   _shuf(b, pk)))

        @plsc.parallel_loop(0, C, 1, unroll=4)
        def edge_body(e):
            t = [kvbuf[e, pl.ds(h * DH, DH)]
                 * qbuf[e, pl.ds(h * DH, DH)]
                 * ebuf[e, pl.ds(h * DH, DH)] for h in range(H)]
            m = [merge(t[2 * j], t[2 * j + 1], 1) for j in range(4)]
            n = [merge(m[2 * j], m[2 * j + 1], 2) for j in range(2)]
            p = merge(n[0], n[1], 4)
            sacc = p + _shuf(p, lane ^ 8)
            score = jnp.exp(jnp.clip(sacc * 0.25, -5.0, 5.0))
            msgbuf[e, pl.ds(D, 16)] = score
            for h in range(H):
                bc = _shuf(score, jnp.full((16,), h, jnp.int32))
                msgbuf[e, pl.ds(h * DH, DH)] = (
                    kvbuf[e, pl.ds(D + h * DH, DH)] * bc)

        pltpu.sync_copy(msgbuf, acc.at[dst_v], add=True)
        return carry
    lax.fori_loop(0, NCHUNK, chunk_body, 0)

    plsc.subcore_barrier()
    out_base = c * NPAD + row0
    pltpu.sync_copy(acc.at[pl.ds(row0, ROWS_PER_TILE)],
                    out.at[pl.ds(out_base, ROWS_PER_TILE)])


def _combine_body(p0_ref, p1_ref, o_ref):
    a = p0_ref[...] + p1_ref[...]
    for h in range(H):
        wv = a[:, h * DH:(h + 1) * DH]
        z = a[:, D + h:D + h + 1]
        o_ref[:, h * DH:(h + 1) * DH] = wv / (z + 1e-6)


def _combine(partials):
    blk = 1024
    grid = (NPAD // blk,)
    return pl.pallas_call(
        _combine_body,
        grid=grid,
        in_specs=[pl.BlockSpec((blk, W), lambda i: (i, 0)),
                  pl.BlockSpec((blk, W), lambda i: (i + NPAD // blk, 0))],
        out_specs=pl.BlockSpec((blk, D), lambda i: (i, 0)),
        out_shape=jax.ShapeDtypeStruct((NPAD, D), jnp.float32),
    )(partials, partials)


def kernel(x, edge_index, edge_attr, WQ, WK, WE, WV):
    qh, kvh = _qkv(x, WQ, WK, WV)
    eh = _eproj(edge_attr, WE)
    src = edge_index[0]
    dst = edge_index[1]
    partials = _sc_attn(kvh, qh, eh, src, dst)
    return _combine(partials)[:N]


# speedup vs baseline: 1.6063x; 1.6063x over previous
import functools

import jax
import jax.numpy as jnp
from jax import lax
from jax.experimental import pallas as pl
from jax.experimental.pallas import tpu as pltpu
from jax.experimental.pallas import tpu_sc as plsc

N = 10000
E = 320000
D = 128
DE = 16
H = 8
DH = 16

NC = 2
NS = 16
NW = NC * NS
PER_TILE = E // NW
C = 40
NCHUNK = PER_TILE // C
W = 144
NPAD = 10240
ROWS_PER_TILE = NPAD // NS


def _qkv_body(x_ref, wq_ref, wk_ref, wv_ref, q_ref, kv_ref):
    xb = x_ref[...]
    q_ref[...] = jnp.dot(xb, wq_ref[...], preferred_element_type=jnp.float32)
    kv_ref[:, :D] = jnp.dot(xb, wk_ref[...], preferred_element_type=jnp.float32)
    kv_ref[:, D:] = jnp.dot(xb, wv_ref[...], preferred_element_type=jnp.float32)


def _qkv(x, WQ, WK, WV):
    blk = 1000
    grid = (N // blk,)
    spec_x = pl.BlockSpec((blk, D), lambda i: (i, 0))
    spec_w = pl.BlockSpec((D, D), lambda i: (0, 0))
    return pl.pallas_call(
        _qkv_body,
        grid=grid,
        in_specs=[spec_x, spec_w, spec_w, spec_w],
        out_specs=[pl.BlockSpec((blk, D), lambda i: (i, 0)),
                   pl.BlockSpec((blk, 2 * D), lambda i: (i, 0))],
        out_shape=[jax.ShapeDtypeStruct((N, D), jnp.float32),
                   jax.ShapeDtypeStruct((N, 2 * D), jnp.float32)],
    )(x, WQ, WK, WV)


def _eproj_body(ea_ref, we_ref, eh_ref):
    eh_ref[...] = jnp.dot(ea_ref[...], we_ref[...],
                          preferred_element_type=jnp.float32)


def _eproj(edge_attr, WE):
    blk = 4000
    grid = (E // blk,)
    return pl.pallas_call(
        _eproj_body,
        grid=grid,
        in_specs=[pl.BlockSpec((blk, DE), lambda i: (i, 0)),
                  pl.BlockSpec((DE, D), lambda i: (0, 0))],
        out_specs=pl.BlockSpec((blk, D), lambda i: (i, 0)),
        out_shape=jax.ShapeDtypeStruct((E, D), jnp.float32),
    )(edge_attr, WE)


_mesh = plsc.VectorSubcoreMesh(core_axis_name="c", subcore_axis_name="s")

_GDN = lax.GatherDimensionNumbers(
    offset_dims=(), collapsed_slice_dims=(0,), start_index_map=(0,))


def _shuf(v, perm):
    return lax.gather(v, perm[:, None], _GDN, (1,),
                      mode=lax.GatherScatterMode.PROMISE_IN_BOUNDS)


@functools.partial(
    pl.kernel,
    out_type=jax.ShapeDtypeStruct((NC * NPAD, W), jnp.float32),
    mesh=_mesh,
    scratch_types=[
        pltpu.VMEM((C,), jnp.int32),
        pltpu.VMEM((C,), jnp.int32),
        pltpu.VMEM((C, 2 * D), jnp.float32),
        pltpu.VMEM((C, D), jnp.float32),
        pltpu.VMEM((C, D), jnp.float32),
        pltpu.VMEM((C, W), jnp.float32),
        pltpu.VMEM_SHARED((NPAD, W), jnp.float32),
        pltpu.SemaphoreType.DMA,
    ],
    compiler_params=pltpu.CompilerParams(use_tc_tiling_on_sc=False),
)
def _sc_attn(kvh, qh, eh, src, dst, out,
             src_v, dst_v, kvbuf, qbuf, ebuf, msgbuf, acc, sem):
    c = lax.axis_index("c")
    s = lax.axis_index("s")
    lane = lax.iota(jnp.int32, 16)
    zero16 = jnp.zeros((16,), jnp.float32)

    def zrow(r, carry):
        for j in range(W // 16):
            msgbuf[r, pl.ds(j * 16, 16)] = zero16
        return carry
    lax.fori_loop(0, C, zrow, 0)
    row0 = s * ROWS_PER_TILE
    for i in range(ROWS_PER_TILE // C):
        pltpu.sync_copy(msgbuf, acc.at[pl.ds(row0 + i * C, C)])
    plsc.subcore_barrier()

    tile_base = (c * NS + s) * PER_TILE

    def chunk_body(i, carry):
        base = tile_base + i * C
        pltpu.sync_copy(src.at[pl.ds(base, C)], src_v)
        pltpu.sync_copy(dst.at[pl.ds(base, C)], dst_v)
        cp1 = pltpu.async_copy(kvh.at[src_v], kvbuf, sem)
        cp2 = pltpu.async_copy(qh.at[dst_v], qbuf, sem)
        cp3 = pltpu.async_copy(eh.at[pl.ds(base, C)], ebuf, sem)
        cp1.wait(); cp2.wait(); cp3.wait()

        def merge(a, b, k):
            m = (lane & k) == 0
            pk = lane ^ k
            return (jnp.where(m, a, b) + jnp.where(m, _shuf(a, pk),
                                                   _shuf(b, pk)))

        @plsc.parallel_loop(0, C, 1, unroll=2)
        def edge_body(e):
            t = [kvbuf[e, pl.ds(h * DH, DH)]
                 * qbuf[e, pl.ds(h * DH, DH)]
                 * ebuf[e, pl.ds(h * DH, DH)] for h in range(H)]
            m = [merge(t[2 * j], t[2 * j + 1], 1) for j in range(4)]
            n = [merge(m[2 * j], m[2 * j + 1], 2) for j in range(2)]
            p = merge(n[0], n[1], 4)
            sacc = p + _shuf(p, lane ^ 8)
            score = jnp.exp(jnp.clip(sacc * 0.25, -5.0, 5.0))
            msgbuf[e, pl.ds(D, 16)] = score
            for h in range(H):
                bc = _shuf(score, jnp.full((16,), h, jnp.int32))
                msgbuf[e, pl.ds(h * DH, DH)] = (
                    kvbuf[e, pl.ds(D + h * DH, DH)] * bc)

        pltpu.sync_copy(msgbuf, acc.at[dst_v], add=True)
        return carry
    lax.fori_loop(0, NCHUNK, chunk_body, 0)

    plsc.subcore_barrier()
    out_base = c * NPAD + row0
    pltpu.sync_copy(acc.at[pl.ds(row0, ROWS_PER_TILE)],
                    out.at[pl.ds(out_base, ROWS_PER_TILE)])


def _combine_body(p0_ref, p1_ref, o_ref):
    a = p0_ref[...] + p1_ref[...]
    for h in range(H):
        wv = a[:, h * DH:(h + 1) * DH]
        z = a[:, D + h:D + h + 1]
        o_ref[:, h * DH:(h + 1) * DH] = wv / (z + 1e-6)


def _combine(partials):
    blk = 1024
    grid = (NPAD // blk,)
    return pl.pallas_call(
        _combine_body,
        grid=grid,
        in_specs=[pl.BlockSpec((blk, W), lambda i: (i, 0)),
                  pl.BlockSpec((blk, W), lambda i: (i + NPAD // blk, 0))],
        out_specs=pl.BlockSpec((blk, D), lambda i: (i, 0)),
        out_shape=jax.ShapeDtypeStruct((NPAD, D), jnp.float32),
    )(partials, partials)


def kernel(x, edge_index, edge_attr, WQ, WK, WE, WV):
    qh, kvh = _qkv(x, WQ, WK, WV)
    eh = _eproj(edge_attr, WE)
    src = edge_index[0]
    dst = edge_index[1]
    partials = _sc_attn(kvh, qh, eh, src, dst)
    return _combine(partials)[:N]


# speedup vs baseline: 1.8674x; 1.1625x over previous
import functools

import jax
import jax.numpy as jnp
from jax import lax
from jax.experimental import pallas as pl
from jax.experimental.pallas import tpu as pltpu
from jax.experimental.pallas import tpu_sc as plsc

N = 10000
E = 320000
D = 128
DE = 16
H = 8
DH = 16

NC = 2
NS = 16
NW = NC * NS
PER_TILE = E // NW
C = 32
W = 144
NPAD = 10240
ROWS_PER_TILE = NPAD // NS


def _qkv_body(x_ref, wq_ref, wk_ref, wv_ref, q_ref, kv_ref):
    xb = x_ref[...]
    q_ref[...] = jnp.dot(xb, wq_ref[...], preferred_element_type=jnp.float32)
    kv_ref[:, :D] = jnp.dot(xb, wk_ref[...], preferred_element_type=jnp.float32)
    kv_ref[:, D:] = jnp.dot(xb, wv_ref[...], preferred_element_type=jnp.float32)


def _qkv(x, WQ, WK, WV):
    blk = 1000
    grid = (N // blk,)
    spec_x = pl.BlockSpec((blk, D), lambda i: (i, 0))
    spec_w = pl.BlockSpec((D, D), lambda i: (0, 0))
    return pl.pallas_call(
        _qkv_body,
        grid=grid,
        in_specs=[spec_x, spec_w, spec_w, spec_w],
        out_specs=[pl.BlockSpec((blk, D), lambda i: (i, 0)),
                   pl.BlockSpec((blk, 2 * D), lambda i: (i, 0))],
        out_shape=[jax.ShapeDtypeStruct((N, D), jnp.float32),
                   jax.ShapeDtypeStruct((N, 2 * D), jnp.float32)],
    )(x, WQ, WK, WV)


def _eproj_body(ea_ref, we_ref, eh_ref):
    eh_ref[...] = jnp.dot(ea_ref[...], we_ref[...],
                          preferred_element_type=jnp.float32)


def _eproj(edge_attr, WE):
    blk = 4000
    grid = (E // blk,)
    return pl.pallas_call(
        _eproj_body,
        grid=grid,
        in_specs=[pl.BlockSpec((blk, DE), lambda i: (i, 0)),
                  pl.BlockSpec((DE, D), lambda i: (0, 0))],
        out_specs=pl.BlockSpec((blk, D), lambda i: (i, 0)),
        out_shape=jax.ShapeDtypeStruct((E, D), jnp.float32),
    )(edge_attr, WE)


_mesh = plsc.VectorSubcoreMesh(core_axis_name="c", subcore_axis_name="s")

_GDN = lax.GatherDimensionNumbers(
    offset_dims=(), collapsed_slice_dims=(0,), start_index_map=(0,))


def _shuf(v, perm):
    return lax.gather(v, perm[:, None], _GDN, (1,),
                      mode=lax.GatherScatterMode.PROMISE_IN_BOUNDS)


NF = PER_TILE // C
CT = PER_TILE - NF * C


@functools.partial(
    pl.kernel,
    out_type=jax.ShapeDtypeStruct((NC * NPAD, W), jnp.float32),
    mesh=_mesh,
    scratch_types=[
        pltpu.VMEM((C,), jnp.int32),
        pltpu.VMEM((C,), jnp.int32),
        pltpu.VMEM((C,), jnp.int32),
        pltpu.VMEM((C,), jnp.int32),
        pltpu.VMEM((CT,), jnp.int32),
        pltpu.VMEM((CT,), jnp.int32),
        pltpu.VMEM((C, 2 * D), jnp.float32),
        pltpu.VMEM((C, 2 * D), jnp.float32),
        pltpu.VMEM((C, D), jnp.float32),
        pltpu.VMEM((C, D), jnp.float32),
        pltpu.VMEM((C, D), jnp.float32),
        pltpu.VMEM((C, D), jnp.float32),
        pltpu.VMEM((C, W), jnp.float32),
        pltpu.VMEM_SHARED((NPAD, W), jnp.float32),
        pltpu.SemaphoreType.DMA,
        pltpu.SemaphoreType.DMA,
    ],
    compiler_params=pltpu.CompilerParams(use_tc_tiling_on_sc=False),
)
def _sc_attn(kvh, qh, eh, src, dst, out,
             srcA, dstA, srcB, dstB, srcT, dstT,
             kvA, kvB, qA, qB, eA, eB, msgbuf, acc, gsemA, gsemB):
    c = lax.axis_index("c")
    s = lax.axis_index("s")
    lane = lax.iota(jnp.int32, 16)
    zero16 = jnp.zeros((16,), jnp.float32)

    def zrow(r, carry):
        for j in range(W // 16):
            msgbuf[r, pl.ds(j * 16, 16)] = zero16
        return carry
    lax.fori_loop(0, C, zrow, 0)
    row0 = s * ROWS_PER_TILE
    for i in range(ROWS_PER_TILE // C):
        pltpu.sync_copy(msgbuf, acc.at[pl.ds(row0 + i * C, C)])
    plsc.subcore_barrier()

    tile_base = (c * NS + s) * PER_TILE

    def merge(a, b, k):
        m = (lane & k) == 0
        pk = lane ^ k
        return (jnp.where(m, a, b) + jnp.where(m, _shuf(a, pk),
                                               _shuf(b, pk)))

    def do_chunk(kvb, qb, eb, dstb, count):
        @plsc.parallel_loop(0, count, 1, unroll=2)
        def edge_body(e):
            t = [kvb[e, pl.ds(h * DH, DH)]
                 * qb[e, pl.ds(h * DH, DH)]
                 * eb[e, pl.ds(h * DH, DH)] for h in range(H)]
            m = [merge(t[2 * j], t[2 * j + 1], 1) for j in range(4)]
            n = [merge(m[2 * j], m[2 * j + 1], 2) for j in range(2)]
            p = merge(n[0], n[1], 4)
            sacc = p + _shuf(p, lane ^ 8)
            score = jnp.exp(jnp.clip(sacc * 0.25, -5.0, 5.0))
            msgbuf[e, pl.ds(D, 16)] = score
            for h in range(H):
                bc = _shuf(score, jnp.full((16,), h, jnp.int32))
                msgbuf[e, pl.ds(h * DH, DH)] = (
                    kvb[e, pl.ds(D + h * DH, DH)] * bc)
        if count == C:
            pltpu.sync_copy(msgbuf, acc.at[dstb], add=True)
        else:
            pltpu.sync_copy(msgbuf.at[pl.ds(0, count)], acc.at[dstb],
                            add=True)

    def load_idx(base, srcb, dstb):
        pltpu.sync_copy(src.at[pl.ds(base, srcb.shape[0])], srcb)
        pltpu.sync_copy(dst.at[pl.ds(base, dstb.shape[0])], dstb)

    def issue(base, srcb, dstb, kvb, qb, eb, sem):
        pltpu.async_copy(kvh.at[srcb], kvb, sem)
        pltpu.async_copy(qh.at[dstb], qb, sem)
        pltpu.async_copy(eh.at[pl.ds(base, kvb.shape[0])], eb, sem)

    def drain(kvb, qb, eb, sem):
        pltpu.make_async_copy(kvh.at[pl.ds(0, kvb.shape[0])], kvb,
                              sem).wait()
        pltpu.make_async_copy(qh.at[pl.ds(0, qb.shape[0])], qb, sem).wait()
        pltpu.make_async_copy(eh.at[pl.ds(0, eb.shape[0])], eb, sem).wait()

    load_idx(tile_base, srcA, dstA)
    issue(tile_base, srcA, dstA, kvA, qA, eA, gsemA)

    def pair_body(j, carry):
        i1 = 2 * j + 1
        base1 = tile_base + i1 * C
        load_idx(base1, srcB, dstB)
        issue(base1, srcB, dstB, kvB, qB, eB, gsemB)
        drain(kvA, qA, eA, gsemA)
        do_chunk(kvA, qA, eA, dstA, C)
        i2 = jnp.minimum(2 * j + 2, NF - 1)
        base2 = tile_base + i2 * C
        load_idx(base2, srcA, dstA)
        issue(base2, srcA, dstA, kvA, qA, eA, gsemA)
        drain(kvB, qB, eB, gsemB)
        do_chunk(kvB, qB, eB, dstB, C)
        return carry
    lax.fori_loop(0, NF // 2, pair_body, 0)

    drain(kvA, qA, eA, gsemA)

    tbase = tile_base + NF * C
    pltpu.sync_copy(src.at[pl.ds(tbase, CT)], srcT)
    pltpu.sync_copy(dst.at[pl.ds(tbase, CT)], dstT)
    pltpu.async_copy(kvh.at[srcT], kvA.at[pl.ds(0, CT)], gsemA)
    pltpu.async_copy(qh.at[dstT], qA.at[pl.ds(0, CT)], gsemA)
    pltpu.async_copy(eh.at[pl.ds(tbase, CT)], eA.at[pl.ds(0, CT)], gsemA)
    pltpu.make_async_copy(kvh.at[pl.ds(0, CT)], kvA.at[pl.ds(0, CT)],
                          gsemA).wait()
    pltpu.make_async_copy(qh.at[pl.ds(0, CT)], qA.at[pl.ds(0, CT)],
                          gsemA).wait()
    pltpu.make_async_copy(eh.at[pl.ds(0, CT)], eA.at[pl.ds(0, CT)],
                          gsemA).wait()
    do_chunk(kvA, qA, eA, dstT, CT)

    plsc.subcore_barrier()
    out_base = c * NPAD + row0
    pltpu.sync_copy(acc.at[pl.ds(row0, ROWS_PER_TILE)],
                    out.at[pl.ds(out_base, ROWS_PER_TILE)])


def _combine_body(p0_ref, p1_ref, o_ref):
    a = p0_ref[...] + p1_ref[...]
    for h in range(H):
        wv = a[:, h * DH:(h + 1) * DH]
        z = a[:, D + h:D + h + 1]
        o_ref[:, h * DH:(h + 1) * DH] = wv / (z + 1e-6)


def _combine(partials):
    blk = 1024
    grid = (NPAD // blk,)
    return pl.pallas_call(
        _combine_body,
        grid=grid,
        in_specs=[pl.BlockSpec((blk, W), lambda i: (i, 0)),
                  pl.BlockSpec((blk, W), lambda i: (i + NPAD // blk, 0))],
        out_specs=pl.BlockSpec((blk, D), lambda i: (i, 0)),
        out_shape=jax.ShapeDtypeStruct((NPAD, D), jnp.float32),
    )(partials, partials)


def kernel(x, edge_index, edge_attr, WQ, WK, WE, WV):
    qh, kvh = _qkv(x, WQ, WK, WV)
    eh = _eproj(edge_attr, WE)
    src = edge_index[0]
    dst = edge_index[1]
    partials = _sc_attn(kvh, qh, eh, src, dst)
    return _combine(partials)[:N]
